# baseline (speedup 1.0000x reference)
import functools

import jax
import jax.numpy as jnp
from jax import lax
from jax.experimental import pallas as pl
from jax.experimental.pallas import tpu as pltpu
from jax.experimental.pallas import tpu_sc as plsc

NEG_SLOPE = 0.2
K = 16
NR = 10240
HALF = NR // 2
NT = 16
TA = HALF // NT
_KNN_NEG = -3.0e38
_SC_PARAMS = pltpu.CompilerParams(needs_layout_passes=False, use_tc_tiling_on_sc=False)


def _knn_kernel(xr_ref, x_ref, xxr_ref, xx_ref, idx_ref):
    inner = -2.0 * jnp.dot(xr_ref[...].T, x_ref[...], preferred_element_type=jnp.float32)
    dist = -xxr_ref[...] - inner - xx_ref[...]
    BR, NP = dist.shape
    iota = lax.broadcasted_iota(jnp.int32, (BR, NP), 1)

    m = jnp.max(dist, axis=1)
    am = jnp.argmax(dist, axis=1).astype(jnp.int32)
    idx_ref[:, 0] = am

    def body(t, carry):
        m, am = carry
        later = (dist < m[:, None]) | ((dist == m[:, None]) & (iota > am[:, None]))
        cand = jnp.where(later, dist, _KNN_NEG)
        m2 = jnp.max(cand, axis=1)
        am2 = jnp.argmax(cand, axis=1).astype(jnp.int32)
        idx_ref[:, t] = am2
        return (m2, am2)

    lax.fori_loop(1, K, body, (m, am), unroll=True)


def _knn(x):
    _, N = x.shape
    BR = 256
    pad = jnp.full((3, NR - N), 1.0e4, jnp.float32)
    xp = jnp.concatenate([x, pad], axis=1)
    xx = jnp.sum(xp * xp, axis=0)
    idx = pl.pallas_call(
        _knn_kernel,
        grid=(NR // BR,),
        in_specs=[
            pl.BlockSpec((3, BR), lambda i: (0, i)),
            pl.BlockSpec((3, NR), lambda i: (0, 0)),
            pl.BlockSpec((BR, 1), lambda i: (i, 0)),
            pl.BlockSpec((1, NR), lambda i: (0, 0)),
        ],
        out_specs=pl.BlockSpec((BR, K), lambda i: (i, 0)),
        out_shape=jax.ShapeDtypeStruct((NR, K), jnp.int32),
    )(xp, xp, xx.reshape(NR, 1), xx.reshape(1, NR))
    return idx


def _proj_kernel(x_ref, w_ref, asrc_ref, adst_ref, h_ref, as_ref, ad_ref, ms_ref, md_ref):
    i = pl.program_id(0)
    h = jnp.dot(x_ref[...], w_ref[...], preferred_element_type=jnp.float32)
    h_ref[...] = h
    a_s = jnp.sum(h * asrc_ref[...], axis=-1, keepdims=True)
    a_d = jnp.sum(h * adst_ref[...], axis=-1, keepdims=True)
    as_ref[...] = a_s
    ad_ref[...] = a_d

    @pl.when(i == 0)
    def _():
        ms_ref[...] = jnp.full((1, 1), -3.0e38, jnp.float32)
        md_ref[...] = jnp.full((1, 1), -3.0e38, jnp.float32)

    ms_ref[...] = jnp.maximum(ms_ref[...], jnp.max(a_s).reshape(1, 1))
    md_ref[...] = jnp.maximum(md_ref[...], jnp.max(a_d).reshape(1, 1))


def _project(x, W, att_src, att_dst):
    N, IN = x.shape
    OUT = W.shape[1]
    BN = 1024
    h, a_s, a_d, ms, md = pl.pallas_call(
        _proj_kernel,
        grid=(N // BN,),
        in_specs=[
            pl.BlockSpec((BN, IN), lambda i: (i, 0)),
            pl.BlockSpec((IN, OUT), lambda i: (0, 0)),
            pl.BlockSpec((1, OUT), lambda i: (0, 0)),
            pl.BlockSpec((1, OUT), lambda i: (0, 0)),
        ],
        out_specs=[
            pl.BlockSpec((BN, OUT), lambda i: (i, 0)),
            pl.BlockSpec((BN, 1), lambda i: (i, 0)),
            pl.BlockSpec((BN, 1), lambda i: (i, 0)),
            pl.BlockSpec((1, 1), lambda i: (0, 0)),
            pl.BlockSpec((1, 1), lambda i: (0, 0)),
        ],
        out_shape=[
            jax.ShapeDtypeStruct((N, OUT), jnp.float32),
            jax.ShapeDtypeStruct((N, 1), jnp.float32),
            jax.ShapeDtypeStruct((N, 1), jnp.float32),
            jax.ShapeDtypeStruct((1, 1), jnp.float32),
            jax.ShapeDtypeStruct((1, 1), jnp.float32),
        ],
    )(x, W, att_src.reshape(1, OUT), att_dst.reshape(1, OUT))
    return h, a_s[:, 0], a_d[:, 0], ms[0, 0], md[0, 0]


def _lrelu(z):
    return jnp.where(z > 0, z, NEG_SLOPE * z)


@jax.jit
def _sc_pq(idx, asrc, adst, mv):
    mesh = plsc.VectorSubcoreMesh(core_axis_name="c", subcore_axis_name="s")
    TB = NR // 32

    @functools.partial(
        pl.kernel,
        out_type=[
            jax.ShapeDtypeStruct((NR, K), jnp.float32),
            jax.ShapeDtypeStruct((NR, K), jnp.float32),
        ],
        mesh=mesh,
        compiler_params=_SC_PARAMS,
        scratch_types=[
            pltpu.VMEM((NR,), jnp.float32),
            pltpu.VMEM((NR,), jnp.float32),
            pltpu.VMEM((TB, 16), jnp.int32),
            pltpu.VMEM((16, 16), jnp.float32),
            pltpu.VMEM((16, 16), jnp.float32),
            pltpu.VMEM((16,), jnp.float32),
        ],
    )
    def pq_kernel(idx_hbm, asrc_hbm, adst_hbm, mv_hbm, p_hbm, q_hbm,
                  asrc_v, adst_v, idxv, ps, qs, mvv):
        c = lax.axis_index("c")
        s = lax.axis_index("s")
        base = pl.multiple_of((c * NT + s) * TB, 64)
        pltpu.sync_copy(asrc_hbm, asrc_v)
        pltpu.sync_copy(adst_hbm, adst_v)
        pltpu.sync_copy(mv_hbm, mvv)
        pltpu.sync_copy(idx_hbm.at[pl.ds(base, TB)], idxv)
        M = mvv[...]

        def body(t, _):
            n = base + t
            t16 = t % 16
            iv = idxv[t]
            nn = jnp.full((16,), n, jnp.int32)
            asg = plsc.load_gather(asrc_v, [iv])
            adg = plsc.load_gather(adst_v, [iv])
            asn = plsc.load_gather(asrc_v, [nn])
            adn = plsc.load_gather(adst_v, [nn])
            ps[t16, pl.ds(0, 16)] = jnp.exp(_lrelu(asg + adn) - M)
            qs[t16, pl.ds(0, 16)] = jnp.exp(_lrelu(asn + adg) - M)

            @pl.when(t16 == 15)
            def _():
                b = pl.multiple_of(base + t - 15, 16)
                pltpu.sync_copy(ps, p_hbm.at[pl.ds(b, 16)])
                pltpu.sync_copy(qs, q_hbm.at[pl.ds(b, 16)])

            return 0

        lax.fori_loop(0, TB, body, 0)

    return pq_kernel(idx, asrc, adst, mv)


@functools.partial(jax.jit, static_argnames=("D",))
def _sc_edge(idx, p, q, hs, bias, D):
    Dh = D // 2
    DW = Dh + 16
    NV = Dh // 16
    TN = NR // NT
    mesh = plsc.VectorSubcoreMesh(core_axis_name="c", subcore_axis_name="s")

    @functools.partial(
        pl.kernel,
        out_type=jax.ShapeDtypeStruct((2 * NR, Dh), jnp.float32),
        mesh=mesh,
        compiler_params=_SC_PARAMS,
        scratch_types=[
            pltpu.VMEM_SHARED((NR, DW), jnp.float32),
            pltpu.VMEM((16, 16), jnp.int32),
            pltpu.VMEM((16, 16), jnp.float32),
            pltpu.VMEM((16, 16), jnp.float32),
            pltpu.VMEM((64, Dh), jnp.float32),
            pltpu.VMEM((32, Dh), jnp.float32),
            pltpu.VMEM((32, DW), jnp.float32),
            pltpu.VMEM((16, DW), jnp.float32),
            pltpu.VMEM((D,), jnp.float32),
            pltpu.SemaphoreType.DMA,
            pltpu.SemaphoreType.DMA,
            pltpu.SemaphoreType.DMA,
            pltpu.SemaphoreType.DMA,
            pltpu.SemaphoreType.DMA,
        ],
    )
    def edge_kernel(idx_hbm, p_hbm, q_hbm, hs_hbm, b_hbm, out_hbm,
                    acc, idxb, pb, qb, hbuf, gbuf, stage, e1buf, bv,
                    gsem0, gsem1, ssem0, ssem1, esem):
        c = lax.axis_index("c")
        s = lax.axis_index("s")
        nb0 = pl.multiple_of(s * TN, 64)
        hoff = c * NR
        iota16 = lax.broadcasted_iota(jnp.int32, (16,), 0)
        one0 = jnp.where(iota16 == 0, 1.0, 0.0)
        zeros16 = jnp.zeros((16,), jnp.float32)
        gsems = (gsem0, gsem1)
        ssems = (ssem0, ssem1)

        pltpu.sync_copy(b_hbm, bv)
        for r in range(16):
            for v in range(DW // 16):
                stage[r, pl.ds(v * 16, 16)] = zeros16
        for z in range(TN // 16):
            pltpu.sync_copy(stage.at[pl.ds(0, 16)], acc.at[pl.ds(nb0 + z * 16, 16)])
        plsc.subcore_barrier()

        def load_chunk(base):
            b = pl.multiple_of(base, 16)
            pltpu.sync_copy(idx_hbm.at[pl.ds(b, 16)], idxb)
            pltpu.sync_copy(p_hbm.at[pl.ds(b, 16)], pb)
            pltpu.sync_copy(q_hbm.at[pl.ds(b, 16)], qb)

        def load_h(base):
            pltpu.sync_copy(hs_hbm.at[pl.ds(pl.multiple_of(hoff + base, 64), 64)], hbuf)

        load_chunk(nb0)
        load_h(nb0)
        pltpu.async_copy(hs_hbm.at[idxb[0] + hoff], gbuf.at[pl.ds(0, 16)], gsem0)

        def body(t2, _):
            for j in range(2):
                t = 2 * t2 + j
                g0 = j * 16
                t16 = t % 16
                iv = idxb[t16]
                pltpu.make_async_copy(hs_hbm.at[iv], gbuf.at[pl.ds(g0, 16)], gsems[j]).wait()
                tt = jnp.full((16,), t16, jnp.int32)
                accv = [zeros16 for _ in range(NV)]
                for k in range(16):
                    pk = plsc.load_gather(pb, [tt, jnp.full((16,), k, jnp.int32)])
                    for v in range(NV):
                        accv[v] = accv[v] + pk * gbuf[g0 + k, pl.ds(v * 16, 16)]

                @pl.when((t16 == 0) & (t >= 16))
                def _():
                    pltpu.make_async_copy(e1buf, acc.at[iv], esem).wait()

                for v in range(NV):
                    e1buf[t16, pl.ds(v * 16, 16)] = accv[v]
                e1buf[t16, pl.ds(Dh, 16)] = jnp.sum(pb[t16]) * one0

                @pl.when(t >= 2)
                def _():
                    pltpu.make_async_copy(stage.at[pl.ds(g0, 16)], acc.at[iv], ssems[j]).wait()

                hv = [hbuf[t % 64, pl.ds(v * 16, 16)] for v in range(NV)]
                for k in range(16):
                    qk = plsc.load_gather(qb, [tt, jnp.full((16,), k, jnp.int32)])
                    for v in range(NV):
                        stage[g0 + k, pl.ds(v * 16, 16)] = qk * hv[v]
                    stage[g0 + k, pl.ds(Dh, 16)] = qk * one0
                pltpu.async_copy(stage.at[pl.ds(g0, 16)], acc.at[iv], ssems[j], add=True)

                @pl.when(t16 == 15)
                def _():
                    idv = (nb0 + t - 15) + iota16
                    pltpu.async_copy(e1buf, acc.at[idv], esem, add=True)

                @pl.when((t16 == 15) & (t < TN - 1))
                def _():
                    load_chunk(nb0 + t + 1)

                @pl.when(((t + 1) % 64 == 0) & (t < TN - 1))
                def _():
                    load_h(nb0 + t + 1)

                @pl.when(t < TN - 1)
                def _():
                    niv = idxb[(t + 1) % 16]
                    pltpu.async_copy(hs_hbm.at[niv + hoff], gbuf.at[pl.ds((1 - j) * 16, 16)],
                                     gsems[1 - j])

            return 0

        lax.fori_loop(0, TN // 2, body, 0)
        dummy = jnp.zeros((16,), jnp.int32)
        pltpu.make_async_copy(stage.at[pl.ds(0, 16)], acc.at[dummy], ssem0).wait()
        pltpu.make_async_copy(stage.at[pl.ds(16, 16)], acc.at[dummy], ssem1).wait()
        pltpu.make_async_copy(e1buf, acc.at[dummy], esem).wait()
        plsc.subcore_barrier()

        bias = [bv[pl.ds(c * Dh + v * 16, 16)] for v in range(NV)]

        def body_f(f, _):
            basel = pl.multiple_of(nb0 + f * 16, 16)
            baseo = pl.multiple_of(hoff + nb0 + f * 16, 16)
            pltpu.sync_copy(acc.at[pl.ds(basel, 16)], stage.at[pl.ds(0, 16)])
            for r in range(16):
                dv = stage[r, pl.ds(Dh, 16)]
                rr = jnp.sum(one0 / (dv + 1e-16))
                for v in range(NV):
                    gbuf[r, pl.ds(v * 16, 16)] = stage[r, pl.ds(v * 16, 16)] * rr + bias[v]
            pltpu.sync_copy(gbuf.at[pl.ds(0, 16)], out_hbm.at[pl.ds(baseo, 16)])
            return 0

        lax.fori_loop(0, TN // 16, body_f, 0)

    return edge_kernel(idx, p, q, hs, bias)


def _gat_layer(x, idx, W, att_src, att_dst, bias):
    h, a_src, a_dst, ms, md = _project(x, W, att_src, att_dst)
    Z = ms + md
    M = jnp.where(Z > 0, Z, NEG_SLOPE * Z)
    mv = jnp.full((16,), M, jnp.float32)
    p, q = _sc_pq(idx, a_src, a_dst, mv)
    D = W.shape[1]
    Dh = D // 2
    hs = jnp.concatenate([h[:, :Dh], h[:, Dh:]], axis=0)
    out2 = _sc_edge(idx, p, q, hs, bias, D=D)
    return jnp.concatenate([out2[:NR], out2[NR:]], axis=1)


def kernel(coordinates, features, W1, a1_src, a1_dst, b1, W2, a2_src, a2_dst, b2, W3, a3_src, a3_dst, b3):
    B, _, N = coordinates.shape
    idx = _knn(coordinates[0])
    x = jnp.transpose(features[0], (1, 0))
    x = jnp.pad(x, ((0, NR - N), (0, 0)))
    g = _gat_layer(x, idx, W1, a1_src, a1_dst, b1)
    g = _gat_layer(g, idx, W2, a2_src, a2_dst, b2)
    g = _gat_layer(g, idx, W3, a3_src, a3_dst, b3)
    return jnp.transpose(g[:N], (1, 0)).reshape(B, -1, N)


# speedup vs baseline: 1.1988x; 1.1988x over previous
import functools

import jax
import jax.numpy as jnp
from jax import lax
from jax.experimental import pallas as pl
from jax.experimental.pallas import tpu as pltpu
from jax.experimental.pallas import tpu_sc as plsc

NEG_SLOPE = 0.2
K = 16
NR = 10240
HALF = NR // 2
NT = 16
TA = HALF // NT
_KNN_NEG = -3.0e38
_SC_PARAMS = pltpu.CompilerParams(needs_layout_passes=False, use_tc_tiling_on_sc=False)


def _knn_kernel(xr_ref, x_ref, xxr_ref, xx_ref, idx_ref):
    inner = -2.0 * jnp.dot(xr_ref[...].T, x_ref[...], preferred_element_type=jnp.float32)
    dist = -xxr_ref[...] - inner - xx_ref[...]
    BR, NP = dist.shape
    iota = lax.broadcasted_iota(jnp.int32, (BR, NP), 1)

    def body(t, dist):
        am = jnp.argmax(dist, axis=1).astype(jnp.int32)
        idx_ref[:, t] = am
        return jnp.where(iota == am[:, None], _KNN_NEG, dist)

    lax.fori_loop(0, K, body, dist, unroll=True)


def _knn(x):
    _, N = x.shape
    BR = 512
    pad = jnp.full((3, NR - N), 1.0e4, jnp.float32)
    xp = jnp.concatenate([x, pad], axis=1)
    xx = jnp.sum(xp * xp, axis=0)
    idx = pl.pallas_call(
        _knn_kernel,
        grid=(NR // BR,),
        in_specs=[
            pl.BlockSpec((3, BR), lambda i: (0, i)),
            pl.BlockSpec((3, NR), lambda i: (0, 0)),
            pl.BlockSpec((BR, 1), lambda i: (i, 0)),
            pl.BlockSpec((1, NR), lambda i: (0, 0)),
        ],
        out_specs=pl.BlockSpec((BR, K), lambda i: (i, 0)),
        out_shape=jax.ShapeDtypeStruct((NR, K), jnp.int32),
    )(xp, xp, xx.reshape(NR, 1), xx.reshape(1, NR))
    return idx


def _proj_kernel(x_ref, w_ref, asrc_ref, adst_ref, h_ref, as_ref, ad_ref, ms_ref, md_ref):
    i = pl.program_id(0)
    h = jnp.dot(x_ref[...], w_ref[...], preferred_element_type=jnp.float32)
    h_ref[...] = h
    a_s = jnp.sum(h * asrc_ref[...], axis=-1, keepdims=True)
    a_d = jnp.sum(h * adst_ref[...], axis=-1, keepdims=True)
    as_ref[...] = a_s
    ad_ref[...] = a_d

    @pl.when(i == 0)
    def _():
        ms_ref[...] = jnp.full((1, 1), -3.0e38, jnp.float32)
        md_ref[...] = jnp.full((1, 1), -3.0e38, jnp.float32)

    ms_ref[...] = jnp.maximum(ms_ref[...], jnp.max(a_s).reshape(1, 1))
    md_ref[...] = jnp.maximum(md_ref[...], jnp.max(a_d).reshape(1, 1))


def _project(x, W, att_src, att_dst):
    N, IN = x.shape
    OUT = W.shape[1]
    BN = 1024
    h, a_s, a_d, ms, md = pl.pallas_call(
        _proj_kernel,
        grid=(N // BN,),
        in_specs=[
            pl.BlockSpec((BN, IN), lambda i: (i, 0)),
            pl.BlockSpec((IN, OUT), lambda i: (0, 0)),
            pl.BlockSpec((1, OUT), lambda i: (0, 0)),
            pl.BlockSpec((1, OUT), lambda i: (0, 0)),
        ],
        out_specs=[
            pl.BlockSpec((BN, OUT), lambda i: (i, 0)),
            pl.BlockSpec((BN, 1), lambda i: (i, 0)),
            pl.BlockSpec((BN, 1), lambda i: (i, 0)),
            pl.BlockSpec((1, 1), lambda i: (0, 0)),
            pl.BlockSpec((1, 1), lambda i: (0, 0)),
        ],
        out_shape=[
            jax.ShapeDtypeStruct((N, OUT), jnp.float32),
            jax.ShapeDtypeStruct((N, 1), jnp.float32),
            jax.ShapeDtypeStruct((N, 1), jnp.float32),
            jax.ShapeDtypeStruct((1, 1), jnp.float32),
            jax.ShapeDtypeStruct((1, 1), jnp.float32),
        ],
    )(x, W, att_src.reshape(1, OUT), att_dst.reshape(1, OUT))
    return h, a_s[:, 0], a_d[:, 0], ms[0, 0], md[0, 0]


def _lrelu(z):
    return jnp.where(z > 0, z, NEG_SLOPE * z)


@jax.jit
def _sc_pq(idx, asrc, adst, mv):
    mesh = plsc.VectorSubcoreMesh(core_axis_name="c", subcore_axis_name="s")
    TB = NR // 32

    @functools.partial(
        pl.kernel,
        out_type=[
            jax.ShapeDtypeStruct((NR, K), jnp.float32),
            jax.ShapeDtypeStruct((NR, K), jnp.float32),
        ],
        mesh=mesh,
        compiler_params=_SC_PARAMS,
        scratch_types=[
            pltpu.VMEM((NR,), jnp.float32),
            pltpu.VMEM((NR,), jnp.float32),
            pltpu.VMEM((TB, 16), jnp.int32),
            pltpu.VMEM((16, 16), jnp.float32),
            pltpu.VMEM((16, 16), jnp.float32),
            pltpu.VMEM((16,), jnp.float32),
        ],
    )
    def pq_kernel(idx_hbm, asrc_hbm, adst_hbm, mv_hbm, p_hbm, q_hbm,
                  asrc_v, adst_v, idxv, ps, qs, mvv):
        c = lax.axis_index("c")
        s = lax.axis_index("s")
        base = pl.multiple_of((c * NT + s) * TB, 64)
        pltpu.sync_copy(asrc_hbm, asrc_v)
        pltpu.sync_copy(adst_hbm, adst_v)
        pltpu.sync_copy(mv_hbm, mvv)
        pltpu.sync_copy(idx_hbm.at[pl.ds(base, TB)], idxv)
        M = mvv[...]

        def body(t, _):
            n = base + t
            t16 = t % 16
            iv = idxv[t]
            nn = jnp.full((16,), n, jnp.int32)
            asg = plsc.load_gather(asrc_v, [iv])
            adg = plsc.load_gather(adst_v, [iv])
            asn = plsc.load_gather(asrc_v, [nn])
            adn = plsc.load_gather(adst_v, [nn])
            ps[t16, pl.ds(0, 16)] = jnp.exp(_lrelu(asg + adn) - M)
            qs[t16, pl.ds(0, 16)] = jnp.exp(_lrelu(asn + adg) - M)

            @pl.when(t16 == 15)
            def _():
                b = pl.multiple_of(base + t - 15, 16)
                pltpu.sync_copy(ps, p_hbm.at[pl.ds(b, 16)])
                pltpu.sync_copy(qs, q_hbm.at[pl.ds(b, 16)])

            return 0

        lax.fori_loop(0, TB, body, 0)

    return pq_kernel(idx, asrc, adst, mv)


@functools.partial(jax.jit, static_argnames=("D",))
def _sc_edge(idx, p, q, hs, bias, D):
    Dh = D // 2
    DW = Dh + 16
    NV = Dh // 16
    TN = NR // NT
    mesh = plsc.VectorSubcoreMesh(core_axis_name="c", subcore_axis_name="s")

    @functools.partial(
        pl.kernel,
        out_type=jax.ShapeDtypeStruct((2 * NR, Dh), jnp.float32),
        mesh=mesh,
        compiler_params=_SC_PARAMS,
        scratch_types=[
            pltpu.VMEM_SHARED((NR, DW), jnp.float32),
            pltpu.VMEM((16, 16), jnp.int32),
            pltpu.VMEM((16, 16), jnp.float32),
            pltpu.VMEM((16, 16), jnp.float32),
            pltpu.VMEM((64, Dh), jnp.float32),
            pltpu.VMEM((32, Dh), jnp.float32),
            pltpu.VMEM((32, DW), jnp.float32),
            pltpu.VMEM((16, DW), jnp.float32),
            pltpu.VMEM((D,), jnp.float32),
            pltpu.SemaphoreType.DMA,
            pltpu.SemaphoreType.DMA,
            pltpu.SemaphoreType.DMA,
            pltpu.SemaphoreType.DMA,
            pltpu.SemaphoreType.DMA,
        ],
    )
    def edge_kernel(idx_hbm, p_hbm, q_hbm, hs_hbm, b_hbm, out_hbm,
                    acc, idxb, pb, qb, hbuf, gbuf, stage, e1buf, bv,
                    gsem0, gsem1, ssem0, ssem1, esem):
        c = lax.axis_index("c")
        s = lax.axis_index("s")
        nb0 = pl.multiple_of(s * TN, 64)
        hoff = c * NR
        iota16 = lax.broadcasted_iota(jnp.int32, (16,), 0)
        one0 = jnp.where(iota16 == 0, 1.0, 0.0)
        zeros16 = jnp.zeros((16,), jnp.float32)
        gsems = (gsem0, gsem1)
        ssems = (ssem0, ssem1)

        pltpu.sync_copy(b_hbm, bv)
        for r in range(16):
            for v in range(DW // 16):
                stage[r, pl.ds(v * 16, 16)] = zeros16
        for z in range(TN // 16):
            pltpu.sync_copy(stage.at[pl.ds(0, 16)], acc.at[pl.ds(nb0 + z * 16, 16)])
        plsc.subcore_barrier()

        def load_chunk(base):
            b = pl.multiple_of(base, 16)
            pltpu.sync_copy(idx_hbm.at[pl.ds(b, 16)], idxb)
            pltpu.sync_copy(p_hbm.at[pl.ds(b, 16)], pb)
            pltpu.sync_copy(q_hbm.at[pl.ds(b, 16)], qb)

        def load_h(base):
            pltpu.sync_copy(hs_hbm.at[pl.ds(pl.multiple_of(hoff + base, 64), 64)], hbuf)

        load_chunk(nb0)
        load_h(nb0)
        pltpu.async_copy(hs_hbm.at[idxb[0] + hoff], gbuf.at[pl.ds(0, 16)], gsem0)

        def body(t2, _):
            for j in range(2):
                t = 2 * t2 + j
                g0 = j * 16
                t16 = t % 16
                iv = idxb[t16]
                pltpu.make_async_copy(hs_hbm.at[iv], gbuf.at[pl.ds(g0, 16)], gsems[j]).wait()
                tt = jnp.full((16,), t16, jnp.int32)
                accv = [zeros16 for _ in range(NV)]
                for k in range(16):
                    pk = plsc.load_gather(pb, [tt, jnp.full((16,), k, jnp.int32)])
                    for v in range(NV):
                        accv[v] = accv[v] + pk * gbuf[g0 + k, pl.ds(v * 16, 16)]

                @pl.when((t16 == 0) & (t >= 16))
                def _():
                    pltpu.make_async_copy(e1buf, acc.at[iv], esem).wait()

                for v in range(NV):
                    e1buf[t16, pl.ds(v * 16, 16)] = accv[v]
                e1buf[t16, pl.ds(Dh, 16)] = jnp.sum(pb[t16]) * one0

                @pl.when(t >= 2)
                def _():
                    pltpu.make_async_copy(stage.at[pl.ds(g0, 16)], acc.at[iv], ssems[j]).wait()

                hv = [hbuf[t % 64, pl.ds(v * 16, 16)] for v in range(NV)]
                for k in range(16):
                    qk = plsc.load_gather(qb, [tt, jnp.full((16,), k, jnp.int32)])
                    for v in range(NV):
                        stage[g0 + k, pl.ds(v * 16, 16)] = qk * hv[v]
                    stage[g0 + k, pl.ds(Dh, 16)] = qk * one0
                pltpu.async_copy(stage.at[pl.ds(g0, 16)], acc.at[iv], ssems[j], add=True)

                @pl.when(t16 == 15)
                def _():
                    idv = (nb0 + t - 15) + iota16
                    pltpu.async_copy(e1buf, acc.at[idv], esem, add=True)

                @pl.when((t16 == 15) & (t < TN - 1))
                def _():
                    load_chunk(nb0 + t + 1)

                @pl.when(((t + 1) % 64 == 0) & (t < TN - 1))
                def _():
                    load_h(nb0 + t + 1)

                @pl.when(t < TN - 1)
                def _():
                    niv = idxb[(t + 1) % 16]
                    pltpu.async_copy(hs_hbm.at[niv + hoff], gbuf.at[pl.ds((1 - j) * 16, 16)],
                                     gsems[1 - j])

            return 0

        lax.fori_loop(0, TN // 2, body, 0)
        dummy = jnp.zeros((16,), jnp.int32)
        pltpu.make_async_copy(stage.at[pl.ds(0, 16)], acc.at[dummy], ssem0).wait()
        pltpu.make_async_copy(stage.at[pl.ds(16, 16)], acc.at[dummy], ssem1).wait()
        pltpu.make_async_copy(e1buf, acc.at[dummy], esem).wait()
        plsc.subcore_barrier()

        bias = [bv[pl.ds(c * Dh + v * 16, 16)] for v in range(NV)]

        def body_f(f, _):
            basel = pl.multiple_of(nb0 + f * 16, 16)
            baseo = pl.multiple_of(hoff + nb0 + f * 16, 16)
            pltpu.sync_copy(acc.at[pl.ds(basel, 16)], stage.at[pl.ds(0, 16)])
            for r in range(16):
                dv = stage[r, pl.ds(Dh, 16)]
                rr = jnp.sum(one0 / (dv + 1e-16))
                for v in range(NV):
                    gbuf[r, pl.ds(v * 16, 16)] = stage[r, pl.ds(v * 16, 16)] * rr + bias[v]
            pltpu.sync_copy(gbuf.at[pl.ds(0, 16)], out_hbm.at[pl.ds(baseo, 16)])
            return 0

        lax.fori_loop(0, TN // 16, body_f, 0)

    return edge_kernel(idx, p, q, hs, bias)


def _gat_layer(x, idx, W, att_src, att_dst, bias):
    h, a_src, a_dst, ms, md = _project(x, W, att_src, att_dst)
    Z = ms + md
    M = jnp.where(Z > 0, Z, NEG_SLOPE * Z)
    mv = jnp.full((16,), M, jnp.float32)
    p, q = _sc_pq(idx, a_src, a_dst, mv)
    D = W.shape[1]
    Dh = D // 2
    hs = jnp.concatenate([h[:, :Dh], h[:, Dh:]], axis=0)
    out2 = _sc_edge(idx, p, q, hs, bias, D=D)
    return jnp.concatenate([out2[:NR], out2[NR:]], axis=1)


def kernel(coordinates, features, W1, a1_src, a1_dst, b1, W2, a2_src, a2_dst, b2, W3, a3_src, a3_dst, b3):
    B, _, N = coordinates.shape
    idx = _knn(coordinates[0])
    x = jnp.transpose(features[0], (1, 0))
    x = jnp.pad(x, ((0, NR - N), (0, 0)))
    g = _gat_layer(x, idx, W1, a1_src, a1_dst, b1)
    g = _gat_layer(g, idx, W2, a2_src, a2_dst, b2)
    g = _gat_layer(g, idx, W3, a3_src, a3_dst, b3)
    return jnp.transpose(g[:N], (1, 0)).reshape(B, -1, N)


# speedup vs baseline: 1.2051x; 1.0052x over previous
import functools

import jax
import jax.numpy as jnp
from jax import lax
from jax.experimental import pallas as pl
from jax.experimental.pallas import tpu as pltpu
from jax.experimental.pallas import tpu_sc as plsc

NEG_SLOPE = 0.2
K = 16
NR = 10240
HALF = NR // 2
NT = 16
TA = HALF // NT
_KNN_NEG = -3.0e38
_SC_PARAMS = pltpu.CompilerParams(needs_layout_passes=False, use_tc_tiling_on_sc=False)


def _knn_kernel(xr_ref, x_ref, xxr_ref, xx_ref, idx_ref):
    inner = -2.0 * jnp.dot(xr_ref[...].T, x_ref[...], preferred_element_type=jnp.float32)
    dist = -xxr_ref[...] - inner - xx_ref[...]
    BR, NP = dist.shape
    iota = lax.broadcasted_iota(jnp.int32, (BR, NP), 1)

    def body(t, dist):
        am = jnp.argmax(dist, axis=1).astype(jnp.int32)
        idx_ref[:, t] = am
        return jnp.where(iota == am[:, None], _KNN_NEG, dist)

    lax.fori_loop(0, K, body, dist, unroll=True)


def _knn(x):
    _, N = x.shape
    BR = 640
    pad = jnp.full((3, NR - N), 1.0e4, jnp.float32)
    xp = jnp.concatenate([x, pad], axis=1)
    xx = jnp.sum(xp * xp, axis=0)
    idx = pl.pallas_call(
        _knn_kernel,
        grid=(NR // BR,),
        in_specs=[
            pl.BlockSpec((3, BR), lambda i: (0, i)),
            pl.BlockSpec((3, NR), lambda i: (0, 0)),
            pl.BlockSpec((BR, 1), lambda i: (i, 0)),
            pl.BlockSpec((1, NR), lambda i: (0, 0)),
        ],
        out_specs=pl.BlockSpec((BR, K), lambda i: (i, 0)),
        out_shape=jax.ShapeDtypeStruct((NR, K), jnp.int32),
    )(xp, xp, xx.reshape(NR, 1), xx.reshape(1, NR))
    return idx


def _proj_kernel(x_ref, w_ref, asrc_ref, adst_ref, h_ref, as_ref, ad_ref, ms_ref, md_ref):
    i = pl.program_id(0)
    h = jnp.dot(x_ref[...], w_ref[...], preferred_element_type=jnp.float32)
    h_ref[...] = h
    a_s = jnp.sum(h * asrc_ref[...], axis=-1, keepdims=True)
    a_d = jnp.sum(h * adst_ref[...], axis=-1, keepdims=True)
    as_ref[...] = a_s
    ad_ref[...] = a_d

    @pl.when(i == 0)
    def _():
        ms_ref[...] = jnp.full((1, 1), -3.0e38, jnp.float32)
        md_ref[...] = jnp.full((1, 1), -3.0e38, jnp.float32)

    ms_ref[...] = jnp.maximum(ms_ref[...], jnp.max(a_s).reshape(1, 1))
    md_ref[...] = jnp.maximum(md_ref[...], jnp.max(a_d).reshape(1, 1))


def _project(x, W, att_src, att_dst):
    N, IN = x.shape
    OUT = W.shape[1]
    BN = 1024
    h, a_s, a_d, ms, md = pl.pallas_call(
        _proj_kernel,
        grid=(N // BN,),
        in_specs=[
            pl.BlockSpec((BN, IN), lambda i: (i, 0)),
            pl.BlockSpec((IN, OUT), lambda i: (0, 0)),
            pl.BlockSpec((1, OUT), lambda i: (0, 0)),
            pl.BlockSpec((1, OUT), lambda i: (0, 0)),
        ],
        out_specs=[
            pl.BlockSpec((BN, OUT), lambda i: (i, 0)),
            pl.BlockSpec((BN, 1), lambda i: (i, 0)),
            pl.BlockSpec((BN, 1), lambda i: (i, 0)),
            pl.BlockSpec((1, 1), lambda i: (0, 0)),
            pl.BlockSpec((1, 1), lambda i: (0, 0)),
        ],
        out_shape=[
            jax.ShapeDtypeStruct((N, OUT), jnp.float32),
            jax.ShapeDtypeStruct((N, 1), jnp.float32),
            jax.ShapeDtypeStruct((N, 1), jnp.float32),
            jax.ShapeDtypeStruct((1, 1), jnp.float32),
            jax.ShapeDtypeStruct((1, 1), jnp.float32),
        ],
    )(x, W, att_src.reshape(1, OUT), att_dst.reshape(1, OUT))
    return h, a_s[:, 0], a_d[:, 0], ms[0, 0], md[0, 0]


def _lrelu(z):
    return jnp.where(z > 0, z, NEG_SLOPE * z)


@jax.jit
def _sc_pq(idx, asrc, adst, mv):
    mesh = plsc.VectorSubcoreMesh(core_axis_name="c", subcore_axis_name="s")
    TB = NR // 32

    @functools.partial(
        pl.kernel,
        out_type=[
            jax.ShapeDtypeStruct((NR, K), jnp.float32),
            jax.ShapeDtypeStruct((NR, K), jnp.float32),
        ],
        mesh=mesh,
        compiler_params=_SC_PARAMS,
        scratch_types=[
            pltpu.VMEM((NR,), jnp.float32),
            pltpu.VMEM((NR,), jnp.float32),
            pltpu.VMEM((TB, 16), jnp.int32),
            pltpu.VMEM((16, 16), jnp.float32),
            pltpu.VMEM((16, 16), jnp.float32),
            pltpu.VMEM((16,), jnp.float32),
        ],
    )
    def pq_kernel(idx_hbm, asrc_hbm, adst_hbm, mv_hbm, p_hbm, q_hbm,
                  asrc_v, adst_v, idxv, ps, qs, mvv):
        c = lax.axis_index("c")
        s = lax.axis_index("s")
        base = pl.multiple_of((c * NT + s) * TB, 64)
        pltpu.sync_copy(asrc_hbm, asrc_v)
        pltpu.sync_copy(adst_hbm, adst_v)
        pltpu.sync_copy(mv_hbm, mvv)
        pltpu.sync_copy(idx_hbm.at[pl.ds(base, TB)], idxv)
        M = mvv[...]

        def body(t, _):
            n = base + t
            t16 = t % 16
            iv = idxv[t]
            nn = jnp.full((16,), n, jnp.int32)
            asg = plsc.load_gather(asrc_v, [iv])
            adg = plsc.load_gather(adst_v, [iv])
            asn = plsc.load_gather(asrc_v, [nn])
            adn = plsc.load_gather(adst_v, [nn])
            ps[t16, pl.ds(0, 16)] = jnp.exp(_lrelu(asg + adn) - M)
            qs[t16, pl.ds(0, 16)] = jnp.exp(_lrelu(asn + adg) - M)

            @pl.when(t16 == 15)
            def _():
                b = pl.multiple_of(base + t - 15, 16)
                pltpu.sync_copy(ps, p_hbm.at[pl.ds(b, 16)])
                pltpu.sync_copy(qs, q_hbm.at[pl.ds(b, 16)])

            return 0

        lax.fori_loop(0, TB, body, 0)

    return pq_kernel(idx, asrc, adst, mv)


@functools.partial(jax.jit, static_argnames=("D",))
def _sc_edge(idx, p, q, hs, bias, D):
    Dh = D // 2
    DW = Dh + 16
    NV = Dh // 16
    TN = NR // NT
    mesh = plsc.VectorSubcoreMesh(core_axis_name="c", subcore_axis_name="s")

    @functools.partial(
        pl.kernel,
        out_type=jax.ShapeDtypeStruct((2 * NR, Dh), jnp.float32),
        mesh=mesh,
        compiler_params=_SC_PARAMS,
        scratch_types=[
            pltpu.VMEM_SHARED((NR, DW), jnp.float32),
            pltpu.VMEM((16, 16), jnp.int32),
            pltpu.VMEM((16, 16), jnp.float32),
            pltpu.VMEM((16, 16), jnp.float32),
            pltpu.VMEM((64, Dh), jnp.float32),
            pltpu.VMEM((32, Dh), jnp.float32),
            pltpu.VMEM((32, DW), jnp.float32),
            pltpu.VMEM((16, DW), jnp.float32),
            pltpu.VMEM((D,), jnp.float32),
            pltpu.SemaphoreType.DMA,
            pltpu.SemaphoreType.DMA,
            pltpu.SemaphoreType.DMA,
            pltpu.SemaphoreType.DMA,
            pltpu.SemaphoreType.DMA,
        ],
    )
    def edge_kernel(idx_hbm, p_hbm, q_hbm, hs_hbm, b_hbm, out_hbm,
                    acc, idxb, pb, qb, hbuf, gbuf, stage, e1buf, bv,
                    gsem0, gsem1, ssem0, ssem1, esem):
        c = lax.axis_index("c")
        s = lax.axis_index("s")
        nb0 = pl.multiple_of(s * TN, 64)
        hoff = c * NR
        iota16 = lax.broadcasted_iota(jnp.int32, (16,), 0)
        one0 = jnp.where(iota16 == 0, 1.0, 0.0)
        zeros16 = jnp.zeros((16,), jnp.float32)
        gsems = (gsem0, gsem1)
        ssems = (ssem0, ssem1)

        pltpu.sync_copy(b_hbm, bv)
        for r in range(16):
            for v in range(DW // 16):
                stage[r, pl.ds(v * 16, 16)] = zeros16
        for z in range(TN // 16):
            pltpu.sync_copy(stage.at[pl.ds(0, 16)], acc.at[pl.ds(nb0 + z * 16, 16)])
        plsc.subcore_barrier()

        def load_chunk(base):
            b = pl.multiple_of(base, 16)
            pltpu.sync_copy(idx_hbm.at[pl.ds(b, 16)], idxb)
            pltpu.sync_copy(p_hbm.at[pl.ds(b, 16)], pb)
            pltpu.sync_copy(q_hbm.at[pl.ds(b, 16)], qb)

        def load_h(base):
            pltpu.sync_copy(hs_hbm.at[pl.ds(pl.multiple_of(hoff + base, 64), 64)], hbuf)

        load_chunk(nb0)
        load_h(nb0)
        pltpu.async_copy(hs_hbm.at[idxb[0] + hoff], gbuf.at[pl.ds(0, 16)], gsem0)

        def body(t2, _):
            for j in range(2):
                t = 2 * t2 + j
                g0 = j * 16
                t16 = t % 16
                iv = idxb[t16]
                pltpu.make_async_copy(hs_hbm.at[iv], gbuf.at[pl.ds(g0, 16)], gsems[j]).wait()
                tt = jnp.full((16,), t16, jnp.int32)
                accv = [zeros16 for _ in range(NV)]
                for k in range(16):
                    pk = plsc.load_gather(pb, [tt, jnp.full((16,), k, jnp.int32)])
                    for v in range(NV):
                        accv[v] = accv[v] + pk * gbuf[g0 + k, pl.ds(v * 16, 16)]

                @pl.when((t16 == 0) & (t >= 16))
                def _():
                    pltpu.make_async_copy(e1buf, acc.at[iv], esem).wait()

                for v in range(NV):
                    e1buf[t16, pl.ds(v * 16, 16)] = accv[v]
                e1buf[t16, pl.ds(Dh, 16)] = jnp.sum(pb[t16]) * one0

                @pl.when(t >= 2)
                def _():
                    pltpu.make_async_copy(stage.at[pl.ds(g0, 16)], acc.at[iv], ssems[j]).wait()

                hv = [hbuf[t % 64, pl.ds(v * 16, 16)] for v in range(NV)]
                for k in range(16):
                    qk = plsc.load_gather(qb, [tt, jnp.full((16,), k, jnp.int32)])
                    for v in range(NV):
                        stage[g0 + k, pl.ds(v * 16, 16)] = qk * hv[v]
                    stage[g0 + k, pl.ds(Dh, 16)] = qk * one0
                pltpu.async_copy(stage.at[pl.ds(g0, 16)], acc.at[iv], ssems[j], add=True)

                @pl.when(t16 == 15)
                def _():
                    idv = (nb0 + t - 15) + iota16
                    pltpu.async_copy(e1buf, acc.at[idv], esem, add=True)

                @pl.when((t16 == 15) & (t < TN - 1))
                def _():
                    load_chunk(nb0 + t + 1)

                @pl.when(((t + 1) % 64 == 0) & (t < TN - 1))
                def _():
                    load_h(nb0 + t + 1)

                @pl.when(t < TN - 1)
                def _():
                    niv = idxb[(t + 1) % 16]
                    pltpu.async_copy(hs_hbm.at[niv + hoff], gbuf.at[pl.ds((1 - j) * 16, 16)],
                                     gsems[1 - j])

            return 0

        lax.fori_loop(0, TN // 2, body, 0)
        dummy = jnp.zeros((16,), jnp.int32)
        pltpu.make_async_copy(stage.at[pl.ds(0, 16)], acc.at[dummy], ssem0).wait()
        pltpu.make_async_copy(stage.at[pl.ds(16, 16)], acc.at[dummy], ssem1).wait()
        pltpu.make_async_copy(e1buf, acc.at[dummy], esem).wait()
        plsc.subcore_barrier()

        bias = [bv[pl.ds(c * Dh + v * 16, 16)] for v in range(NV)]

        def body_f(f, _):
            basel = pl.multiple_of(nb0 + f * 16, 16)
            baseo = pl.multiple_of(hoff + nb0 + f * 16, 16)
            pltpu.sync_copy(acc.at[pl.ds(basel, 16)], stage.at[pl.ds(0, 16)])
            for r in range(16):
                dv = stage[r, pl.ds(Dh, 16)]
                rr = jnp.sum(one0 / (dv + 1e-16))
                for v in range(NV):
                    gbuf[r, pl.ds(v * 16, 16)] = stage[r, pl.ds(v * 16, 16)] * rr + bias[v]
            pltpu.sync_copy(gbuf.at[pl.ds(0, 16)], out_hbm.at[pl.ds(baseo, 16)])
            return 0

        lax.fori_loop(0, TN // 16, body_f, 0)

    return edge_kernel(idx, p, q, hs, bias)


def _gat_layer(x, idx, W, att_src, att_dst, bias):
    h, a_src, a_dst, ms, md = _project(x, W, att_src, att_dst)
    Z = ms + md
    M = jnp.where(Z > 0, Z, NEG_SLOPE * Z)
    mv = jnp.full((16,), M, jnp.float32)
    p, q = _sc_pq(idx, a_src, a_dst, mv)
    D = W.shape[1]
    Dh = D // 2
    hs = jnp.concatenate([h[:, :Dh], h[:, Dh:]], axis=0)
    out2 = _sc_edge(idx, p, q, hs, bias, D=D)
    return jnp.concatenate([out2[:NR], out2[NR:]], axis=1)


def kernel(coordinates, features, W1, a1_src, a1_dst, b1, W2, a2_src, a2_dst, b2, W3, a3_src, a3_dst, b3):
    B, _, N = coordinates.shape
    idx = _knn(coordinates[0])
    x = jnp.transpose(features[0], (1, 0))
    x = jnp.pad(x, ((0, NR - N), (0, 0)))
    g = _gat_layer(x, idx, W1, a1_src, a1_dst, b1)
    g = _gat_layer(g, idx, W2, a2_src, a2_dst, b2)
    g = _gat_layer(g, idx, W3, a3_src, a3_dst, b3)
    return jnp.transpose(g[:N], (1, 0)).reshape(B, -1, N)


# speedup vs baseline: 1.3437x; 1.1150x over previous
import functools

import jax
import jax.numpy as jnp
from jax import lax
from jax.experimental import pallas as pl
from jax.experimental.pallas import tpu as pltpu
from jax.experimental.pallas import tpu_sc as plsc

NEG_SLOPE = 0.2
K = 16
NR = 10240
HALF = NR // 2
NT = 16
TA = HALF // NT
_KNN_NEG = -3.0e38
_SC_PARAMS = pltpu.CompilerParams(needs_layout_passes=False, use_tc_tiling_on_sc=False)


def _knn_kernel(xr_ref, x_ref, xxr_ref, xx_ref, idx_ref):
    inner = -2.0 * jnp.dot(xr_ref[...].T, x_ref[...], preferred_element_type=jnp.float32)
    dist = -xxr_ref[...] - inner - xx_ref[...]
    BR, NP = dist.shape
    iota = lax.broadcasted_iota(jnp.int32, (BR, NP), 1)

    def body(t, dist):
        am = jnp.argmax(dist, axis=1).astype(jnp.int32)
        idx_ref[:, t] = am
        return jnp.where(iota == am[:, None], _KNN_NEG, dist)

    lax.fori_loop(0, K, body, dist, unroll=True)


def _knn(x):
    _, N = x.shape
    BR = 640
    pad = jnp.full((3, NR - N), 1.0e4, jnp.float32)
    xp = jnp.concatenate([x, pad], axis=1)
    xx = jnp.sum(xp * xp, axis=0)
    idx = pl.pallas_call(
        _knn_kernel,
        grid=(NR // BR,),
        in_specs=[
            pl.BlockSpec((3, BR), lambda i: (0, i)),
            pl.BlockSpec((3, NR), lambda i: (0, 0)),
            pl.BlockSpec((BR, 1), lambda i: (i, 0)),
            pl.BlockSpec((1, NR), lambda i: (0, 0)),
        ],
        out_specs=pl.BlockSpec((BR, K), lambda i: (i, 0)),
        out_shape=jax.ShapeDtypeStruct((NR, K), jnp.int32),
    )(xp, xp, xx.reshape(NR, 1), xx.reshape(1, NR))
    return idx


def _proj_kernel(x_ref, w_ref, asrc_ref, adst_ref, h_ref, as_ref, ad_ref, ms_ref, md_ref):
    i = pl.program_id(0)
    h = jnp.dot(x_ref[...], w_ref[...], preferred_element_type=jnp.float32)
    h_ref[...] = h
    a_s = jnp.sum(h * asrc_ref[...], axis=-1, keepdims=True)
    a_d = jnp.sum(h * adst_ref[...], axis=-1, keepdims=True)
    as_ref[...] = a_s
    ad_ref[...] = a_d

    @pl.when(i == 0)
    def _():
        ms_ref[...] = jnp.full((1, 1), -3.0e38, jnp.float32)
        md_ref[...] = jnp.full((1, 1), -3.0e38, jnp.float32)

    ms_ref[...] = jnp.maximum(ms_ref[...], jnp.max(a_s).reshape(1, 1))
    md_ref[...] = jnp.maximum(md_ref[...], jnp.max(a_d).reshape(1, 1))


def _project(x, W, att_src, att_dst):
    N, IN = x.shape
    OUT = W.shape[1]
    BN = 1024
    h, a_s, a_d, ms, md = pl.pallas_call(
        _proj_kernel,
        grid=(N // BN,),
        in_specs=[
            pl.BlockSpec((BN, IN), lambda i: (i, 0)),
            pl.BlockSpec((IN, OUT), lambda i: (0, 0)),
            pl.BlockSpec((1, OUT), lambda i: (0, 0)),
            pl.BlockSpec((1, OUT), lambda i: (0, 0)),
        ],
        out_specs=[
            pl.BlockSpec((BN, OUT), lambda i: (i, 0)),
            pl.BlockSpec((BN, 1), lambda i: (i, 0)),
            pl.BlockSpec((BN, 1), lambda i: (i, 0)),
            pl.BlockSpec((1, 1), lambda i: (0, 0)),
            pl.BlockSpec((1, 1), lambda i: (0, 0)),
        ],
        out_shape=[
            jax.ShapeDtypeStruct((N, OUT), jnp.float32),
            jax.ShapeDtypeStruct((N, 1), jnp.float32),
            jax.ShapeDtypeStruct((N, 1), jnp.float32),
            jax.ShapeDtypeStruct((1, 1), jnp.float32),
            jax.ShapeDtypeStruct((1, 1), jnp.float32),
        ],
    )(x, W, att_src.reshape(1, OUT), att_dst.reshape(1, OUT))
    return h, a_s[:, 0], a_d[:, 0], ms[0, 0], md[0, 0]


def _lrelu(z):
    return jnp.where(z > 0, z, NEG_SLOPE * z)


@jax.jit
def _sc_pq(idx, asrc, adst, mv):
    mesh = plsc.VectorSubcoreMesh(core_axis_name="c", subcore_axis_name="s")
    TB = NR // 32

    @functools.partial(
        pl.kernel,
        out_type=[
            jax.ShapeDtypeStruct((NR, K), jnp.float32),
            jax.ShapeDtypeStruct((NR, K), jnp.float32),
        ],
        mesh=mesh,
        compiler_params=_SC_PARAMS,
        scratch_types=[
            pltpu.VMEM((NR,), jnp.float32),
            pltpu.VMEM((NR,), jnp.float32),
            pltpu.VMEM((TB, 16), jnp.int32),
            pltpu.VMEM((16, 16), jnp.float32),
            pltpu.VMEM((16, 16), jnp.float32),
            pltpu.VMEM((16,), jnp.float32),
        ],
    )
    def pq_kernel(idx_hbm, asrc_hbm, adst_hbm, mv_hbm, p_hbm, q_hbm,
                  asrc_v, adst_v, idxv, ps, qs, mvv):
        c = lax.axis_index("c")
        s = lax.axis_index("s")
        base = pl.multiple_of((c * NT + s) * TB, 64)
        pltpu.sync_copy(asrc_hbm, asrc_v)
        pltpu.sync_copy(adst_hbm, adst_v)
        pltpu.sync_copy(mv_hbm, mvv)
        pltpu.sync_copy(idx_hbm.at[pl.ds(base, TB)], idxv)
        M = mvv[...]

        def body(t, _):
            n = base + t
            t16 = t % 16
            iv = idxv[t]
            nn = jnp.full((16,), n, jnp.int32)
            asg = plsc.load_gather(asrc_v, [iv])
            adg = plsc.load_gather(adst_v, [iv])
            asn = plsc.load_gather(asrc_v, [nn])
            adn = plsc.load_gather(adst_v, [nn])
            ps[t16, pl.ds(0, 16)] = jnp.exp(_lrelu(asg + adn) - M)
            qs[t16, pl.ds(0, 16)] = jnp.exp(_lrelu(asn + adg) - M)

            @pl.when(t16 == 15)
            def _():
                b = pl.multiple_of(base + t - 15, 16)
                pltpu.sync_copy(ps, p_hbm.at[pl.ds(b, 16)])
                pltpu.sync_copy(qs, q_hbm.at[pl.ds(b, 16)])

            return 0

        lax.fori_loop(0, TB, body, 0)

    return pq_kernel(idx, asrc, adst, mv)


@functools.partial(jax.jit, static_argnames=("D",))
def _sc_edge(idx, p, q, hs, bias, D):
    Dh = D // 2
    DW = Dh + 16
    NV = Dh // 16
    TN = NR // NT
    mesh = plsc.VectorSubcoreMesh(core_axis_name="c", subcore_axis_name="s")

    @functools.partial(
        pl.kernel,
        out_type=jax.ShapeDtypeStruct((2 * NR, Dh), jnp.float32),
        mesh=mesh,
        compiler_params=_SC_PARAMS,
        scratch_types=[
            pltpu.VMEM_SHARED((NR, DW), jnp.float32),
            pltpu.VMEM((16, 16), jnp.int32),
            pltpu.VMEM((16, 16), jnp.float32),
            pltpu.VMEM((16, 16), jnp.float32),
            pltpu.VMEM((64, Dh), jnp.float32),
            pltpu.VMEM((32, Dh), jnp.float32),
            pltpu.VMEM((32, DW), jnp.float32),
            pltpu.VMEM((16, DW), jnp.float32),
            pltpu.VMEM((D,), jnp.float32),
            pltpu.SemaphoreType.DMA,
            pltpu.SemaphoreType.DMA,
            pltpu.SemaphoreType.DMA,
            pltpu.SemaphoreType.DMA,
            pltpu.SemaphoreType.DMA,
        ],
    )
    def edge_kernel(idx_hbm, p_hbm, q_hbm, hs_hbm, b_hbm, out_hbm,
                    acc, idxb, pb, qb, hbuf, gbuf, stage, e1buf, bv,
                    gsem0, gsem1, ssem0, ssem1, esem):
        c = lax.axis_index("c")
        s = lax.axis_index("s")
        nb0 = pl.multiple_of(s * TN, 64)
        hoff = c * NR
        iota16 = lax.broadcasted_iota(jnp.int32, (16,), 0)
        one0 = jnp.where(iota16 == 0, 1.0, 0.0)
        zeros16 = jnp.zeros((16,), jnp.float32)
        gsems = (gsem0, gsem1)
        ssems = (ssem0, ssem1)

        pltpu.sync_copy(b_hbm, bv)
        for r in range(16):
            for v in range(DW // 16):
                stage[r, pl.ds(v * 16, 16)] = zeros16
        for z in range(TN // 16):
            pltpu.sync_copy(stage.at[pl.ds(0, 16)], acc.at[pl.ds(nb0 + z * 16, 16)])
        plsc.subcore_barrier()

        def load_chunk(base):
            b = pl.multiple_of(base, 16)
            pltpu.sync_copy(idx_hbm.at[pl.ds(b, 16)], idxb)
            pltpu.sync_copy(p_hbm.at[pl.ds(b, 16)], pb)
            pltpu.sync_copy(q_hbm.at[pl.ds(b, 16)], qb)

        def load_h(base):
            pltpu.sync_copy(hs_hbm.at[pl.ds(pl.multiple_of(hoff + base, 64), 64)], hbuf)

        load_chunk(nb0)
        load_h(nb0)
        pltpu.async_copy(hs_hbm.at[idxb[0] + hoff], gbuf.at[pl.ds(0, 16)], gsem0)

        def body(t2, _):
            for j in range(2):
                t = 2 * t2 + j
                g0 = j * 16
                t16 = t % 16
                iv = idxb[t16]
                pltpu.make_async_copy(hs_hbm.at[iv], gbuf.at[pl.ds(g0, 16)], gsems[j]).wait()

                @pl.when((t16 < 15) & (t < TN - 1))
                def _():
                    niv = idxb[t16 + 1]
                    pltpu.async_copy(hs_hbm.at[niv + hoff], gbuf.at[pl.ds((1 - j) * 16, 16)],
                                     gsems[1 - j])

                tt = jnp.full((16,), t16, jnp.int32)
                accv = [zeros16 for _ in range(NV)]
                for k in range(16):
                    pk = plsc.load_gather(pb, [tt, jnp.full((16,), k, jnp.int32)])
                    for v in range(NV):
                        accv[v] = accv[v] + pk * gbuf[g0 + k, pl.ds(v * 16, 16)]

                @pl.when((t16 == 0) & (t >= 16))
                def _():
                    pltpu.make_async_copy(e1buf, acc.at[iv], esem).wait()

                for v in range(NV):
                    e1buf[t16, pl.ds(v * 16, 16)] = accv[v]
                e1buf[t16, pl.ds(Dh, 16)] = jnp.sum(pb[t16]) * one0

                @pl.when(t >= 2)
                def _():
                    pltpu.make_async_copy(stage.at[pl.ds(g0, 16)], acc.at[iv], ssems[j]).wait()

                hv = [hbuf[t % 64, pl.ds(v * 16, 16)] for v in range(NV)]
                for k in range(16):
                    qk = plsc.load_gather(qb, [tt, jnp.full((16,), k, jnp.int32)])
                    for v in range(NV):
                        stage[g0 + k, pl.ds(v * 16, 16)] = qk * hv[v]
                    stage[g0 + k, pl.ds(Dh, 16)] = qk * one0
                pltpu.async_copy(stage.at[pl.ds(g0, 16)], acc.at[iv], ssems[j], add=True)

                @pl.when(t16 == 15)
                def _():
                    idv = (nb0 + t - 15) + iota16
                    pltpu.async_copy(e1buf, acc.at[idv], esem, add=True)

                @pl.when((t16 == 15) & (t < TN - 1))
                def _():
                    load_chunk(nb0 + t + 1)

                @pl.when(((t + 1) % 64 == 0) & (t < TN - 1))
                def _():
                    load_h(nb0 + t + 1)

                @pl.when((t16 == 15) & (t < TN - 1))
                def _():
                    niv = idxb[0]
                    pltpu.async_copy(hs_hbm.at[niv + hoff], gbuf.at[pl.ds((1 - j) * 16, 16)],
                                     gsems[1 - j])

            return 0

        lax.fori_loop(0, TN // 2, body, 0)
        dummy = jnp.zeros((16,), jnp.int32)
        pltpu.make_async_copy(stage.at[pl.ds(0, 16)], acc.at[dummy], ssem0).wait()
        pltpu.make_async_copy(stage.at[pl.ds(16, 16)], acc.at[dummy], ssem1).wait()
        pltpu.make_async_copy(e1buf, acc.at[dummy], esem).wait()
        plsc.subcore_barrier()

        bias = [bv[pl.ds(c * Dh + v * 16, 16)] for v in range(NV)]

        def body_f(f, _):
            basel = pl.multiple_of(nb0 + f * 16, 16)
            baseo = pl.multiple_of(hoff + nb0 + f * 16, 16)
            pltpu.sync_copy(acc.at[pl.ds(basel, 16)], stage.at[pl.ds(0, 16)])
            for r in range(16):
                dv = stage[r, pl.ds(Dh, 16)]
                rr = jnp.sum(one0 / (dv + 1e-16))
                for v in range(NV):
                    gbuf[r, pl.ds(v * 16, 16)] = stage[r, pl.ds(v * 16, 16)] * rr + bias[v]
            pltpu.sync_copy(gbuf.at[pl.ds(0, 16)], out_hbm.at[pl.ds(baseo, 16)])
            return 0

        lax.fori_loop(0, TN // 16, body_f, 0)

    return edge_kernel(idx, p, q, hs, bias)


def _gat_layer(x, idx, W, att_src, att_dst, bias):
    h, a_src, a_dst, ms, md = _project(x, W, att_src, att_dst)
    Z = ms + md
    M = jnp.where(Z > 0, Z, NEG_SLOPE * Z)
    mv = jnp.full((16,), M, jnp.float32)
    p, q = _sc_pq(idx, a_src, a_dst, mv)
    D = W.shape[1]
    Dh = D // 2
    hs = jnp.concatenate([h[:, :Dh], h[:, Dh:]], axis=0)
    out2 = _sc_edge(idx, p, q, hs, bias, D=D)
    return jnp.concatenate([out2[:NR], out2[NR:]], axis=1)


def kernel(coordinates, features, W1, a1_src, a1_dst, b1, W2, a2_src, a2_dst, b2, W3, a3_src, a3_dst, b3):
    B, _, N = coordinates.shape
    idx = _knn(coordinates[0])
    x = jnp.transpose(features[0], (1, 0))
    x = jnp.pad(x, ((0, NR - N), (0, 0)))
    g = _gat_layer(x, idx, W1, a1_src, a1_dst, b1)
    g = _gat_layer(g, idx, W2, a2_src, a2_dst, b2)
    g = _gat_layer(g, idx, W3, a3_src, a3_dst, b3)
    return jnp.transpose(g[:N], (1, 0)).reshape(B, -1, N)


# speedup vs baseline: 1.5579x; 1.1594x over previous
import functools

import jax
import jax.numpy as jnp
from jax import lax
from jax.experimental import pallas as pl
from jax.experimental.pallas import tpu as pltpu
from jax.experimental.pallas import tpu_sc as plsc

NEG_SLOPE = 0.2
K = 16
NR = 10240
HALF = NR // 2
NT = 16
TA = HALF // NT
_KNN_NEG = -3.0e38
_SC_PARAMS = pltpu.CompilerParams(needs_layout_passes=False, use_tc_tiling_on_sc=False)


def _knn_kernel(xr_ref, x_ref, xxr_ref, xx_ref, idx_ref):
    inner = -2.0 * jnp.dot(xr_ref[...].T, x_ref[...], preferred_element_type=jnp.float32)
    dist = -xxr_ref[...] - inner - xx_ref[...]
    BR, NP = dist.shape
    iota = lax.broadcasted_iota(jnp.int32, (BR, NP), 1)

    def body(t, dist):
        am = jnp.argmax(dist, axis=1).astype(jnp.int32)
        idx_ref[:, t] = am
        return jnp.where(iota == am[:, None], _KNN_NEG, dist)

    lax.fori_loop(0, K, body, dist, unroll=True)


def _knn(x):
    _, N = x.shape
    BR = 640
    pad = jnp.full((3, NR - N), 1.0e4, jnp.float32)
    xp = jnp.concatenate([x, pad], axis=1)
    xx = jnp.sum(xp * xp, axis=0)
    idx = pl.pallas_call(
        _knn_kernel,
        grid=(NR // BR,),
        in_specs=[
            pl.BlockSpec((3, BR), lambda i: (0, i)),
            pl.BlockSpec((3, NR), lambda i: (0, 0)),
            pl.BlockSpec((BR, 1), lambda i: (i, 0)),
            pl.BlockSpec((1, NR), lambda i: (0, 0)),
        ],
        out_specs=pl.BlockSpec((BR, K), lambda i: (i, 0)),
        out_shape=jax.ShapeDtypeStruct((NR, K), jnp.int32),
    )(xp, xp, xx.reshape(NR, 1), xx.reshape(1, NR))
    return idx


def _proj_kernel(x_ref, w_ref, asrc_ref, adst_ref, h_ref, as_ref, ad_ref, ms_ref, md_ref):
    i = pl.program_id(0)
    h = jnp.dot(x_ref[...], w_ref[...], preferred_element_type=jnp.float32)
    h_ref[...] = h
    a_s = jnp.sum(h * asrc_ref[...], axis=-1, keepdims=True)
    a_d = jnp.sum(h * adst_ref[...], axis=-1, keepdims=True)
    as_ref[...] = a_s
    ad_ref[...] = a_d

    @pl.when(i == 0)
    def _():
        ms_ref[...] = jnp.full((1, 1), -3.0e38, jnp.float32)
        md_ref[...] = jnp.full((1, 1), -3.0e38, jnp.float32)

    ms_ref[...] = jnp.maximum(ms_ref[...], jnp.max(a_s).reshape(1, 1))
    md_ref[...] = jnp.maximum(md_ref[...], jnp.max(a_d).reshape(1, 1))


def _project(x, W, att_src, att_dst):
    N, IN = x.shape
    OUT = W.shape[1]
    BN = 1024
    h, a_s, a_d, ms, md = pl.pallas_call(
        _proj_kernel,
        grid=(N // BN,),
        in_specs=[
            pl.BlockSpec((BN, IN), lambda i: (i, 0)),
            pl.BlockSpec((IN, OUT), lambda i: (0, 0)),
            pl.BlockSpec((1, OUT), lambda i: (0, 0)),
            pl.BlockSpec((1, OUT), lambda i: (0, 0)),
        ],
        out_specs=[
            pl.BlockSpec((BN, OUT), lambda i: (i, 0)),
            pl.BlockSpec((BN, 1), lambda i: (i, 0)),
            pl.BlockSpec((BN, 1), lambda i: (i, 0)),
            pl.BlockSpec((1, 1), lambda i: (0, 0)),
            pl.BlockSpec((1, 1), lambda i: (0, 0)),
        ],
        out_shape=[
            jax.ShapeDtypeStruct((N, OUT), jnp.float32),
            jax.ShapeDtypeStruct((N, 1), jnp.float32),
            jax.ShapeDtypeStruct((N, 1), jnp.float32),
            jax.ShapeDtypeStruct((1, 1), jnp.float32),
            jax.ShapeDtypeStruct((1, 1), jnp.float32),
        ],
    )(x, W, att_src.reshape(1, OUT), att_dst.reshape(1, OUT))
    return h, a_s[:, 0], a_d[:, 0], ms[0, 0], md[0, 0]


def _lrelu(z):
    return jnp.where(z > 0, z, NEG_SLOPE * z)


@jax.jit
def _sc_pq(idx, asrc, adst, mv):
    mesh = plsc.VectorSubcoreMesh(core_axis_name="c", subcore_axis_name="s")
    TB = NR // 32

    @functools.partial(
        pl.kernel,
        out_type=[
            jax.ShapeDtypeStruct((NR, K), jnp.float32),
            jax.ShapeDtypeStruct((NR, K), jnp.float32),
        ],
        mesh=mesh,
        compiler_params=_SC_PARAMS,
        scratch_types=[
            pltpu.VMEM((NR,), jnp.float32),
            pltpu.VMEM((NR,), jnp.float32),
            pltpu.VMEM((TB, 16), jnp.int32),
            pltpu.VMEM((16, 16), jnp.float32),
            pltpu.VMEM((16, 16), jnp.float32),
            pltpu.VMEM((16,), jnp.float32),
        ],
    )
    def pq_kernel(idx_hbm, asrc_hbm, adst_hbm, mv_hbm, p_hbm, q_hbm,
                  asrc_v, adst_v, idxv, ps, qs, mvv):
        c = lax.axis_index("c")
        s = lax.axis_index("s")
        base = pl.multiple_of((c * NT + s) * TB, 64)
        pltpu.sync_copy(asrc_hbm, asrc_v)
        pltpu.sync_copy(adst_hbm, adst_v)
        pltpu.sync_copy(mv_hbm, mvv)
        pltpu.sync_copy(idx_hbm.at[pl.ds(base, TB)], idxv)
        M = mvv[...]

        def body(t, _):
            n = base + t
            t16 = t % 16
            iv = idxv[t]
            nn = jnp.full((16,), n, jnp.int32)
            asg = plsc.load_gather(asrc_v, [iv])
            adg = plsc.load_gather(adst_v, [iv])
            asn = plsc.load_gather(asrc_v, [nn])
            adn = plsc.load_gather(adst_v, [nn])
            ps[t16, pl.ds(0, 16)] = jnp.exp(_lrelu(asg + adn) - M)
            qs[t16, pl.ds(0, 16)] = jnp.exp(_lrelu(asn + adg) - M)

            @pl.when(t16 == 15)
            def _():
                b = pl.multiple_of(base + t - 15, 16)
                pltpu.sync_copy(ps, p_hbm.at[pl.ds(b, 16)])
                pltpu.sync_copy(qs, q_hbm.at[pl.ds(b, 16)])

            return 0

        lax.fori_loop(0, TB, body, 0)

    return pq_kernel(idx, asrc, adst, mv)


@functools.partial(jax.jit, static_argnames=("D",))
def _sc_edge(idx, p, q, hs, bias, D):
    Dh = D // 2
    DW = Dh + 16
    NV = Dh // 16
    TN = NR // NT
    mesh = plsc.VectorSubcoreMesh(core_axis_name="c", subcore_axis_name="s")

    @functools.partial(
        pl.kernel,
        out_type=jax.ShapeDtypeStruct((2 * NR, Dh), jnp.float32),
        mesh=mesh,
        compiler_params=_SC_PARAMS,
        scratch_types=[
            pltpu.VMEM_SHARED((NR, DW), jnp.float32),
            pltpu.VMEM((16, 16), jnp.int32),
            pltpu.VMEM((16, 16), jnp.float32),
            pltpu.VMEM((16, 16), jnp.float32),
            pltpu.VMEM((64, Dh), jnp.float32),
            pltpu.VMEM((32, Dh), jnp.float32),
            pltpu.VMEM((32, DW), jnp.float32),
            pltpu.VMEM((16, DW), jnp.float32),
            pltpu.VMEM((D,), jnp.float32),
            pltpu.SemaphoreType.DMA,
            pltpu.SemaphoreType.DMA,
            pltpu.SemaphoreType.DMA,
            pltpu.SemaphoreType.DMA,
            pltpu.SemaphoreType.DMA,
        ],
    )
    def edge_kernel(idx_hbm, p_hbm, q_hbm, hs_hbm, b_hbm, out_hbm,
                    acc, idxb, pb, qb, hbuf, gbuf, stage, e1buf, bv,
                    gsem0, gsem1, ssem0, ssem1, esem):
        c = lax.axis_index("c")
        s = lax.axis_index("s")
        nb0 = pl.multiple_of(s * TN, 64)
        hoff = c * NR
        iota16 = lax.broadcasted_iota(jnp.int32, (16,), 0)
        one0 = jnp.where(iota16 == 0, 1.0, 0.0)
        zeros16 = jnp.zeros((16,), jnp.float32)
        gsems = (gsem0, gsem1)
        ssems = (ssem0, ssem1)

        pltpu.sync_copy(b_hbm, bv)
        for r in range(16):
            for v in range(DW // 16):
                stage[r, pl.ds(v * 16, 16)] = zeros16
        for z in range(TN // 16):
            pltpu.sync_copy(stage.at[pl.ds(0, 16)], acc.at[pl.ds(nb0 + z * 16, 16)])
        plsc.subcore_barrier()

        def load_chunk(base):
            b = pl.multiple_of(base, 16)
            pltpu.sync_copy(idx_hbm.at[pl.ds(b, 16)], idxb)
            pltpu.sync_copy(p_hbm.at[pl.ds(b, 16)], pb)
            pltpu.sync_copy(q_hbm.at[pl.ds(b, 16)], qb)

        def load_h(base):
            pltpu.sync_copy(hs_hbm.at[pl.ds(pl.multiple_of(hoff + base, 64), 64)], hbuf)

        load_chunk(nb0)
        load_h(nb0)
        pltpu.async_copy(hs_hbm.at[idxb[0] + hoff], gbuf.at[pl.ds(0, 16)], gsem0)

        def body(t2, _):
            for j in range(2):
                t = 2 * t2 + j
                g0 = j * 16
                t16 = t % 16
                iv = idxb[t16]
                tt = jnp.full((16,), t16, jnp.int32)

                @pl.when((t16 < 15) & (t < TN - 1))
                def _():
                    niv = idxb[t16 + 1]
                    pltpu.async_copy(hs_hbm.at[niv + hoff], gbuf.at[pl.ds((1 - j) * 16, 16)],
                                     gsems[1 - j])

                @pl.when(t >= 2)
                def _():
                    pltpu.make_async_copy(stage.at[pl.ds(g0, 16)], acc.at[iv], ssems[j]).wait()

                hv = [hbuf[t % 64, pl.ds(v * 16, 16)] for v in range(NV)]
                for k in range(16):
                    qk = plsc.load_gather(qb, [tt, jnp.full((16,), k, jnp.int32)])
                    for v in range(NV):
                        stage[g0 + k, pl.ds(v * 16, 16)] = qk * hv[v]
                    stage[g0 + k, pl.ds(Dh, 16)] = qk * one0
                pltpu.async_copy(stage.at[pl.ds(g0, 16)], acc.at[iv], ssems[j], add=True)

                pltpu.make_async_copy(hs_hbm.at[iv], gbuf.at[pl.ds(g0, 16)], gsems[j]).wait()
                accv = [zeros16 for _ in range(NV)]
                for k in range(16):
                    pk = plsc.load_gather(pb, [tt, jnp.full((16,), k, jnp.int32)])
                    for v in range(NV):
                        accv[v] = accv[v] + pk * gbuf[g0 + k, pl.ds(v * 16, 16)]

                @pl.when((t16 == 0) & (t >= 16))
                def _():
                    pltpu.make_async_copy(e1buf, acc.at[iv], esem).wait()

                for v in range(NV):
                    e1buf[t16, pl.ds(v * 16, 16)] = accv[v]
                e1buf[t16, pl.ds(Dh, 16)] = jnp.sum(pb[t16]) * one0

                @pl.when(t16 == 15)
                def _():
                    idv = (nb0 + t - 15) + iota16
                    pltpu.async_copy(e1buf, acc.at[idv], esem, add=True)

                @pl.when((t16 == 15) & (t < TN - 1))
                def _():
                    load_chunk(nb0 + t + 1)

                @pl.when(((t + 1) % 64 == 0) & (t < TN - 1))
                def _():
                    load_h(nb0 + t + 1)

                @pl.when((t16 == 15) & (t < TN - 1))
                def _():
                    niv = idxb[0]
                    pltpu.async_copy(hs_hbm.at[niv + hoff], gbuf.at[pl.ds((1 - j) * 16, 16)],
                                     gsems[1 - j])

            return 0

        lax.fori_loop(0, TN // 2, body, 0)
        dummy = jnp.zeros((16,), jnp.int32)
        pltpu.make_async_copy(stage.at[pl.ds(0, 16)], acc.at[dummy], ssem0).wait()
        pltpu.make_async_copy(stage.at[pl.ds(16, 16)], acc.at[dummy], ssem1).wait()
        pltpu.make_async_copy(e1buf, acc.at[dummy], esem).wait()
        plsc.subcore_barrier()

        bias = [bv[pl.ds(c * Dh + v * 16, 16)] for v in range(NV)]

        def body_f(f, _):
            basel = pl.multiple_of(nb0 + f * 16, 16)
            baseo = pl.multiple_of(hoff + nb0 + f * 16, 16)
            pltpu.sync_copy(acc.at[pl.ds(basel, 16)], stage.at[pl.ds(0, 16)])
            for r in range(16):
                dv = stage[r, pl.ds(Dh, 16)]
                rr = jnp.sum(one0 / (dv + 1e-16))
                for v in range(NV):
                    gbuf[r, pl.ds(v * 16, 16)] = stage[r, pl.ds(v * 16, 16)] * rr + bias[v]
            pltpu.sync_copy(gbuf.at[pl.ds(0, 16)], out_hbm.at[pl.ds(baseo, 16)])
            return 0

        lax.fori_loop(0, TN // 16, body_f, 0)

    return edge_kernel(idx, p, q, hs, bias)


def _gat_layer(x, idx, W, att_src, att_dst, bias):
    h, a_src, a_dst, ms, md = _project(x, W, att_src, att_dst)
    Z = ms + md
    M = jnp.where(Z > 0, Z, NEG_SLOPE * Z)
    mv = jnp.full((16,), M, jnp.float32)
    p, q = _sc_pq(idx, a_src, a_dst, mv)
    D = W.shape[1]
    Dh = D // 2
    hs = jnp.concatenate([h[:, :Dh], h[:, Dh:]], axis=0)
    out2 = _sc_edge(idx, p, q, hs, bias, D=D)
    return jnp.concatenate([out2[:NR], out2[NR:]], axis=1)


def kernel(coordinates, features, W1, a1_src, a1_dst, b1, W2, a2_src, a2_dst, b2, W3, a3_src, a3_dst, b3):
    B, _, N = coordinates.shape
    idx = _knn(coordinates[0])
    x = jnp.transpose(features[0], (1, 0))
    x = jnp.pad(x, ((0, NR - N), (0, 0)))
    g = _gat_layer(x, idx, W1, a1_src, a1_dst, b1)
    g = _gat_layer(g, idx, W2, a2_src, a2_dst, b2)
    g = _gat_layer(g, idx, W3, a3_src, a3_dst, b3)
    return jnp.transpose(g[:N], (1, 0)).reshape(B, -1, N)
